# Initial kernel scaffold; baseline (speedup 1.0000x reference)
import jax
import jax.numpy as jnp
from jax.experimental import pallas as pl


def kernel(dense_x, lS_o, lS_i, emb_tables, bot_Ws, bot_bs, top_Ws, top_bs):
    raise NotImplementedError("write your pallas kernel here")


# speedup vs baseline: 2.4712x; 2.4712x over previous
import functools
import numpy as np
import jax
import jax.numpy as jnp
from jax import lax
from jax.experimental import pallas as pl
from jax.experimental.pallas import tpu as pltpu

_B = 4096
_EMB = 64
_NT = 26
_VOCAB = 100000
_NF = _NT + 1
_PAIR_I = np.array([i for i in range(_NF) for j in range(i)], dtype=np.int32)
_PAIR_J = np.array([j for i in range(_NF) for j in range(i)], dtype=np.int32)
_NPAIR = _PAIR_I.size


def _sel_matrix():
    S = np.zeros((_NF * _NF, _NPAIR), dtype=np.float32)
    for p in range(_NPAIR):
        S[_PAIR_I[p] * _NF + _PAIR_J[p], p] = 1.0
    return S


def _sc_embedding_bag(emb_flat, idx_flat, seg_flat, zeros):
    from jax.experimental.pallas import tpu_sc as plsc

    info = plsc.get_sparse_core_info()
    NC, NS = info.num_cores, info.num_subcores
    tables_per_core = _NT // NC
    rows_per_sub = _B // NS
    CH = 128
    n_ch = rows_per_sub // CH
    mesh = plsc.VectorSubcoreMesh(core_axis_name="c", subcore_axis_name="s",
                                  num_cores=NC)

    @functools.partial(
        pl.kernel, mesh=mesh,
        compiler_params=pltpu.CompilerParams(use_tc_tiling_on_sc=False),
        out_type=jax.ShapeDtypeStruct((_NT, _B, _EMB), jnp.float32),
        scratch_types=[
            pltpu.VMEM((CH,), jnp.int32),
            pltpu.VMEM((CH,), jnp.int32),
            pltpu.VMEM((CH, _EMB), jnp.float32),
            pltpu.VMEM_SHARED((_B, _EMB), jnp.float32),
        ],
    )
    def sc_k(emb_hbm, idx_hbm, seg_hbm, zero_hbm, out_hbm,
             idx_v, seg_v, rows_v, acc):
        cid = lax.axis_index("c")
        sid = lax.axis_index("s")
        my_lo = sid * rows_per_sub
        for ti in range(tables_per_core):
            t = ti * NC + cid
            pltpu.sync_copy(zero_hbm.at[pl.ds(my_lo, rows_per_sub)],
                            acc.at[pl.ds(my_lo, rows_per_sub)])
            plsc.subcore_barrier()
            for j in range(n_ch):
                base = t * _B + my_lo + j * CH
                pltpu.sync_copy(idx_hbm.at[pl.ds(base, CH)], idx_v)
                pltpu.sync_copy(seg_hbm.at[pl.ds(base, CH)], seg_v)
                pltpu.sync_copy(emb_hbm.at[idx_v], rows_v)
                pltpu.sync_copy(rows_v, acc.at[seg_v], add=True)
            plsc.subcore_barrier()
            pltpu.sync_copy(acc.at[pl.ds(my_lo, rows_per_sub)],
                            out_hbm.at[t].at[pl.ds(my_lo, rows_per_sub)])
            plsc.subcore_barrier()

    return sc_k(emb_flat, idx_flat, seg_flat, zeros)


def _tc_body(xr, lyr, w0r, b0r, w1r, b1r, w2r, b2r,
             t0xr, t0zr, tb0r, t1r, tb1r, t2r, tb2r, t3r, tb3r, t4r, tb4r,
             outr):
    f32 = jnp.float32
    x = xr[...]
    h = jnp.maximum(jnp.dot(x, w0r[...], preferred_element_type=f32) + b0r[...], 0.0)
    h = jnp.maximum(jnp.dot(h, w1r[...], preferred_element_type=f32) + b1r[...], 0.0)
    xb = jnp.dot(h, w2r[...], preferred_element_type=f32) + b2r[...]
    T = jnp.concatenate([xb[:, None, :], lyr[...]], axis=1)
    rows = [jnp.sum(T * T[:, n:n + 1, :], axis=2) for n in range(_NF)]
    Zall = jnp.concatenate(rows, axis=1)
    r = jnp.maximum(jnp.dot(xb, t0xr[...], preferred_element_type=f32)
                    + jnp.dot(Zall, t0zr[...], preferred_element_type=f32)
                    + tb0r[...], 0.0)
    r = jnp.maximum(jnp.dot(r, t1r[...], preferred_element_type=f32) + tb1r[...], 0.0)
    r = jnp.maximum(jnp.dot(r, t2r[...], preferred_element_type=f32) + tb2r[...], 0.0)
    r = jnp.maximum(jnp.dot(r, t3r[...], preferred_element_type=f32) + tb3r[...], 0.0)
    outr[...] = jnp.dot(r, t4r[...], preferred_element_type=f32) + tb4r[...]


def _tc_forward(dense_x, ly_t, wmats, Bb=512):
    grid = (_B // Bb,)

    def full(a):
        spec = pl.BlockSpec(a.shape, lambda i: tuple(0 for _ in a.shape))
        return spec

    in_specs = [
        pl.BlockSpec((Bb, dense_x.shape[1]), lambda i: (i, 0)),
        pl.BlockSpec((Bb, _NT, _EMB), lambda i: (i, 0, 0)),
    ] + [full(w) for w in wmats]
    return pl.pallas_call(
        _tc_body,
        grid=grid,
        in_specs=in_specs,
        out_specs=pl.BlockSpec((Bb, 1), lambda i: (i, 0)),
        out_shape=jax.ShapeDtypeStruct((_B, 1), jnp.float32),
    )(dense_x, ly_t, *wmats)


def kernel(dense_x, lS_o, lS_i, emb_tables, bot_Ws, bot_bs, top_Ws, top_bs):
    idx_flat = (lS_i.astype(jnp.int32)
                + (jnp.arange(_NT, dtype=jnp.int32) * _VOCAB)[:, None]).reshape(-1)
    pos = jnp.arange(_B, dtype=jnp.int32)
    seg = jax.vmap(
        lambda o: jnp.searchsorted(o.astype(jnp.int32), pos, side="right")
    )(lS_o).astype(jnp.int32) - 1
    seg_flat = seg.reshape(-1)
    emb_flat = emb_tables.reshape(_NT * _VOCAB, _EMB)
    zeros = jnp.zeros((_B, _EMB), jnp.float32)

    ly = _sc_embedding_bag(emb_flat, idx_flat, seg_flat, zeros)
    ly_t = jnp.transpose(ly, (1, 0, 2))

    w0, w1, w2 = (w.T for w in bot_Ws)
    b0, b1, b2 = (b[None, :] for b in bot_bs)
    S = jnp.asarray(_sel_matrix())
    t0 = top_Ws[0]
    t0x = t0[:, :_EMB].T
    t0z = S @ t0[:, _EMB:].T
    t1, t2, t3, t4 = (w.T for w in top_Ws[1:])
    tb0, tb1, tb2, tb3, tb4 = (b[None, :] for b in top_bs)
    wmats = [w0, b0, w1, b1, w2, b2,
             t0x, t0z, tb0, t1, tb1, t2, tb2, t3, tb3, t4, tb4]

    return _tc_forward(dense_x, ly_t, wmats)


# speedup vs baseline: 2.8355x; 1.1474x over previous
import functools
import numpy as np
import jax
import jax.numpy as jnp
from jax import lax
from jax.experimental import pallas as pl
from jax.experimental.pallas import tpu as pltpu

_B = 4096
_EMB = 64
_NT = 26
_VOCAB = 100000
_NF = _NT + 1
_PAIR_I = np.array([i for i in range(_NF) for j in range(i)], dtype=np.int32)
_PAIR_J = np.array([j for i in range(_NF) for j in range(i)], dtype=np.int32)
_NPAIR = _PAIR_I.size


def _sel_matrix():
    S = np.zeros((_NF * _NF, _NPAIR), dtype=np.float32)
    for p in range(_NPAIR):
        S[_PAIR_I[p] * _NF + _PAIR_J[p], p] = 1.0
    return S


def _sc_embedding_bag(emb_tables, idx_flat, seg_flat, zeros):
    from jax.experimental.pallas import tpu_sc as plsc

    info = plsc.get_sparse_core_info()
    NC, NS = info.num_cores, info.num_subcores
    tables_per_core = _NT // NC
    rows_per_sub = _B // NS
    CH = 128
    n_ch = rows_per_sub // CH
    mesh = plsc.VectorSubcoreMesh(core_axis_name="c", subcore_axis_name="s",
                                  num_cores=NC)

    @functools.partial(
        pl.kernel, mesh=mesh,
        compiler_params=pltpu.CompilerParams(use_tc_tiling_on_sc=False),
        out_type=jax.ShapeDtypeStruct((_B, _NT, _EMB), jnp.float32),
        scratch_types=[
            pltpu.VMEM((CH,), jnp.int32),
            pltpu.VMEM((CH,), jnp.int32),
            pltpu.VMEM((CH, _EMB), jnp.float32),
            pltpu.VMEM_SHARED((_B, _EMB), jnp.float32),
        ],
    )
    def sc_k(emb_hbm, idx_hbm, seg_hbm, zero_hbm, out_hbm,
             idx_v, seg_v, rows_v, acc):
        cid = lax.axis_index("c")
        sid = lax.axis_index("s")
        my_lo = sid * rows_per_sub
        for ti in range(tables_per_core):
            t = ti * NC + cid
            pltpu.sync_copy(zero_hbm.at[pl.ds(my_lo, rows_per_sub)],
                            acc.at[pl.ds(my_lo, rows_per_sub)])
            plsc.subcore_barrier()
            for j in range(n_ch):
                base = t * _B + my_lo + j * CH
                pltpu.sync_copy(idx_hbm.at[pl.ds(base, CH)], idx_v)
                pltpu.sync_copy(seg_hbm.at[pl.ds(base, CH)], seg_v)
                pltpu.sync_copy(emb_hbm.at[t].at[idx_v], rows_v)
                pltpu.sync_copy(rows_v, acc.at[seg_v], add=True)
            plsc.subcore_barrier()
            pltpu.sync_copy(acc.at[pl.ds(my_lo, rows_per_sub)],
                            out_hbm.at[pl.ds(my_lo, rows_per_sub), t])
            plsc.subcore_barrier()

    return sc_k(emb_tables, idx_flat, seg_flat, zeros)


def _tc_body(xr, lyr, w0r, b0r, w1r, b1r, w2r, b2r,
             t0xr, t0zr, tb0r, t1r, tb1r, t2r, tb2r, t3r, tb3r, t4r, tb4r,
             outr):
    f32 = jnp.float32
    x = xr[...]
    h = jnp.maximum(jnp.dot(x, w0r[...], preferred_element_type=f32) + b0r[...], 0.0)
    h = jnp.maximum(jnp.dot(h, w1r[...], preferred_element_type=f32) + b1r[...], 0.0)
    xb = jnp.dot(h, w2r[...], preferred_element_type=f32) + b2r[...]
    T = jnp.concatenate([xb[:, None, :], lyr[...]], axis=1)
    rows = [jnp.sum(T * T[:, n:n + 1, :], axis=2) for n in range(_NF)]
    Zall = jnp.concatenate(rows, axis=1)
    r = jnp.maximum(jnp.dot(xb, t0xr[...], preferred_element_type=f32)
                    + jnp.dot(Zall, t0zr[...], preferred_element_type=f32)
                    + tb0r[...], 0.0)
    r = jnp.maximum(jnp.dot(r, t1r[...], preferred_element_type=f32) + tb1r[...], 0.0)
    r = jnp.maximum(jnp.dot(r, t2r[...], preferred_element_type=f32) + tb2r[...], 0.0)
    r = jnp.maximum(jnp.dot(r, t3r[...], preferred_element_type=f32) + tb3r[...], 0.0)
    outr[...] = jnp.dot(r, t4r[...], preferred_element_type=f32) + tb4r[...]


def _tc_forward(dense_x, ly_t, wmats, Bb=512):
    grid = (_B // Bb,)

    def full(a):
        spec = pl.BlockSpec(a.shape, lambda i: tuple(0 for _ in a.shape))
        return spec

    in_specs = [
        pl.BlockSpec((Bb, dense_x.shape[1]), lambda i: (i, 0)),
        pl.BlockSpec((Bb, _NT, _EMB), lambda i: (i, 0, 0)),
    ] + [full(w) for w in wmats]
    return pl.pallas_call(
        _tc_body,
        grid=grid,
        in_specs=in_specs,
        out_specs=pl.BlockSpec((Bb, 1), lambda i: (i, 0)),
        out_shape=jax.ShapeDtypeStruct((_B, 1), jnp.float32),
    )(dense_x, ly_t, *wmats)


def kernel(dense_x, lS_o, lS_i, emb_tables, bot_Ws, bot_bs, top_Ws, top_bs):
    idx_flat = lS_i.astype(jnp.int32).reshape(-1)
    hist = jnp.zeros((_NT, _B), jnp.int32).at[
        jnp.arange(_NT, dtype=jnp.int32)[:, None], lS_o.astype(jnp.int32)
    ].add(1)
    seg_flat = (jnp.cumsum(hist, axis=1) - 1).reshape(-1)
    zeros = jnp.zeros((_B, _EMB), jnp.float32)

    ly_t = _sc_embedding_bag(emb_tables, idx_flat, seg_flat, zeros)

    w0, w1, w2 = (w.T for w in bot_Ws)
    b0, b1, b2 = (b[None, :] for b in bot_bs)
    S = jnp.asarray(_sel_matrix())
    t0 = top_Ws[0]
    t0x = t0[:, :_EMB].T
    t0z = S @ t0[:, _EMB:].T
    t1, t2, t3, t4 = (w.T for w in top_Ws[1:])
    tb0, tb1, tb2, tb3, tb4 = (b[None, :] for b in top_bs)
    wmats = [w0, b0, w1, b1, w2, b2,
             t0x, t0z, tb0, t1, tb1, t2, tb2, t3, tb3, t4, tb4]

    return _tc_forward(dense_x, ly_t, wmats)
